# Initial kernel scaffold; baseline (speedup 1.0000x reference)
import jax
import jax.numpy as jnp
from jax.experimental import pallas as pl


def kernel(src_node_ids, dst_node_ids, node_interact_times, src_padded_nodes_neighbor_ids, src_padded_nodes_edge_ids, src_padded_nodes_neighbor_times, dst_padded_nodes_neighbor_ids, dst_padded_nodes_edge_ids, dst_padded_nodes_neighbor_times, node_raw_features, edge_raw_features, W, b):
    raise NotImplementedError("write your pallas kernel here")


# speedup vs baseline: 2.0175x; 2.0175x over previous
import functools

import jax
import jax.numpy as jnp
from jax import lax
from jax.experimental import pallas as pl
from jax.experimental.pallas import tpu as pltpu
from jax.experimental.pallas import tpu_sc as plsc

B = 16384
L = 32
D_FEAT = 128
D_EDGE = 16
T_DIM = 128

NC = 2
NS = 16
NW = NC * NS
ROWS_PER_W = 2 * B // NW
GROUP = 32
GROUPS = ROWS_PER_W // GROUP
CSUB = 4
SUB = GROUP // CSUB


def _sc_body(node_tab, edge_tab, nbr_ids, eid_ids, cur_ids,
             node_sum, edge_sum, cur_rows,
             nbr_idx_v, eid_idx_v, cur_idx_v,
             node_rows_v, edge_rows_v, node_acc_v, edge_acc_v, cur_rows_v,
             sem_n, sem_e, sem_c):
    wid = lax.axis_index("s") * NC + lax.axis_index("c")
    base = wid * ROWS_PER_W

    def group_body(g, carry):
        gbase = base + g * GROUP
        pltpu.sync_copy(cur_ids.at[pl.ds(gbase, GROUP)], cur_idx_v)
        cur_cp = pltpu.async_copy(node_tab.at[cur_idx_v], cur_rows_v, sem_c)

        def sub_body(s, c2):
            row0 = gbase + s * CSUB
            sbase = row0 * L
            pltpu.sync_copy(nbr_ids.at[pl.ds(sbase, CSUB * L)], nbr_idx_v)
            pltpu.sync_copy(eid_ids.at[pl.ds(sbase, CSUB * L)], eid_idx_v)
            cp_n = pltpu.async_copy(node_tab.at[nbr_idx_v], node_rows_v, sem_n)
            cp_e = pltpu.async_copy(edge_tab.at[eid_idx_v], edge_rows_v, sem_e)
            cp_n.wait()
            cp_e.wait()
            for bl in range(CSUB):
                r0 = bl * L
                for v in range(D_FEAT // 16):
                    col = pl.ds(v * 16, 16)
                    acc = node_rows_v[r0, col]
                    for l in range(1, L):
                        acc = acc + node_rows_v[r0 + l, col]
                    node_acc_v[bl, col] = acc
                eacc = edge_rows_v[r0, :]
                for l in range(1, L):
                    eacc = eacc + edge_rows_v[r0 + l, :]
                edge_acc_v[bl, :] = eacc
            pltpu.sync_copy(node_acc_v, node_sum.at[pl.ds(row0, CSUB)])
            pltpu.sync_copy(edge_acc_v, edge_sum.at[pl.ds(row0, CSUB)])
            return c2

        lax.fori_loop(0, SUB, sub_body, 0)
        cur_cp.wait()
        pltpu.sync_copy(cur_rows_v, cur_rows.at[pl.ds(gbase, GROUP)])
        return carry

    lax.fori_loop(0, GROUPS, group_body, 0)


@jax.jit
def _sc_gather(node_tab, edge_tab, nbr_ids, eid_ids, cur_ids):
    mesh = plsc.VectorSubcoreMesh(core_axis_name="c", subcore_axis_name="s")
    f = pl.kernel(
        _sc_body,
        out_type=[
            jax.ShapeDtypeStruct((2 * B, D_FEAT), jnp.float32),
            jax.ShapeDtypeStruct((2 * B, D_EDGE), jnp.float32),
            jax.ShapeDtypeStruct((2 * B, D_FEAT), jnp.float32),
        ],
        mesh=mesh,
        scratch_types=[
            pltpu.VMEM((CSUB * L,), jnp.int32),
            pltpu.VMEM((CSUB * L,), jnp.int32),
            pltpu.VMEM((GROUP,), jnp.int32),
            pltpu.VMEM((CSUB * L, D_FEAT), jnp.float32),
            pltpu.VMEM((CSUB * L, D_EDGE), jnp.float32),
            pltpu.VMEM((CSUB, D_FEAT), jnp.float32),
            pltpu.VMEM((CSUB, D_EDGE), jnp.float32),
            pltpu.VMEM((GROUP, D_FEAT), jnp.float32),
            pltpu.SemaphoreType.DMA,
            pltpu.SemaphoreType.DMA,
            pltpu.SemaphoreType.DMA,
        ],
        compiler_params=pltpu.CompilerParams(use_tc_tiling_on_sc=False),
    )
    return f(node_tab, edge_tab, nbr_ids, eid_ids, cur_ids)


TB = 512


def _tc_body(t_ref, pt_ref, ns_ref, es_ref, cr_ref, wt_time_ref,
             wc_ref, wn_ref, we_ref, wtt_ref, b_ref, o_ref):
    dt = t_ref[...] - pt_ref[...]
    w_time = wt_time_ref[...]
    ts = jnp.cos(dt[:, 0:1] * w_time)
    for l in range(1, L):
        ts = ts + jnp.cos(dt[:, l:l + 1] * w_time)
    acc = jnp.dot(cr_ref[...], wc_ref[...], preferred_element_type=jnp.float32)
    acc = acc + jnp.dot(ns_ref[...], wn_ref[...], preferred_element_type=jnp.float32)
    acc = acc + jnp.dot(es_ref[...], we_ref[...], preferred_element_type=jnp.float32)
    acc = acc + jnp.dot(ts, wtt_ref[...], preferred_element_type=jnp.float32)
    o_ref[...] = jnp.maximum(acc + b_ref[...], 0.0)


@jax.jit
def _tc_fuse(t2, pt2, node_sum, edge_sum, cur_rows, w_time, WcT, WnT, WeT, WtT, bias):
    grid = (2 * B // TB,)
    return pl.pallas_call(
        _tc_body,
        grid=grid,
        in_specs=[
            pl.BlockSpec((TB, 1), lambda i: (i, 0)),
            pl.BlockSpec((TB, L), lambda i: (i, 0)),
            pl.BlockSpec((TB, D_FEAT), lambda i: (i, 0)),
            pl.BlockSpec((TB, D_EDGE), lambda i: (i, 0)),
            pl.BlockSpec((TB, D_FEAT), lambda i: (i, 0)),
            pl.BlockSpec((1, T_DIM), lambda i: (0, 0)),
            pl.BlockSpec((D_FEAT, D_FEAT), lambda i: (0, 0)),
            pl.BlockSpec((D_FEAT, D_FEAT), lambda i: (0, 0)),
            pl.BlockSpec((D_EDGE, D_FEAT), lambda i: (0, 0)),
            pl.BlockSpec((T_DIM, D_FEAT), lambda i: (0, 0)),
            pl.BlockSpec((1, D_FEAT), lambda i: (0, 0)),
        ],
        out_specs=pl.BlockSpec((TB, D_FEAT), lambda i: (i, 0)),
        out_shape=jax.ShapeDtypeStruct((2 * B, D_FEAT), jnp.float32),
    )(t2, pt2, node_sum, edge_sum, cur_rows, w_time, WcT, WnT, WeT, WtT, bias)


def kernel(src_node_ids, dst_node_ids, node_interact_times,
           src_padded_nodes_neighbor_ids, src_padded_nodes_edge_ids, src_padded_nodes_neighbor_times,
           dst_padded_nodes_neighbor_ids, dst_padded_nodes_edge_ids, dst_padded_nodes_neighbor_times,
           node_raw_features, edge_raw_features, W, b):
    nbr2 = jnp.concatenate([src_padded_nodes_neighbor_ids.reshape(-1),
                            dst_padded_nodes_neighbor_ids.reshape(-1)]).astype(jnp.int32)
    eid2 = jnp.concatenate([src_padded_nodes_edge_ids.reshape(-1),
                            dst_padded_nodes_edge_ids.reshape(-1)]).astype(jnp.int32)
    cur2 = jnp.concatenate([src_node_ids, dst_node_ids]).astype(jnp.int32)
    t2 = jnp.concatenate([node_interact_times, node_interact_times])[:, None]
    pt2 = jnp.concatenate([src_padded_nodes_neighbor_times,
                           dst_padded_nodes_neighbor_times], axis=0)

    node_sum, edge_sum, cur_rows = _sc_gather(
        node_raw_features, edge_raw_features, nbr2, eid2, cur2)

    inv_l = jnp.float32(1.0 / L)
    WcT = W[:, :D_FEAT].T
    WnT = (W[:, D_FEAT:2 * D_FEAT] * inv_l).T
    WeT = (W[:, 2 * D_FEAT:2 * D_FEAT + D_EDGE] * inv_l).T
    WtT = (W[:, 2 * D_FEAT + D_EDGE:] * inv_l).T
    bias = b[None, :]
    w_time = (1.0 / (10.0 ** jnp.linspace(0.0, 9.0, T_DIM))).astype(jnp.float32)[None, :]

    out = _tc_fuse(t2, pt2, node_sum, edge_sum, cur_rows,
                   w_time, WcT, WnT, WeT, WtT, bias)
    src_emb = out[:B]
    dst_emb = out[B:]
    return (src_emb, dst_emb, jnp.zeros_like(src_emb))


# speedup vs baseline: 5.2140x; 2.5844x over previous
import functools

import numpy as np
import jax
import jax.numpy as jnp
from jax import lax
from jax.experimental import pallas as pl
from jax.experimental.pallas import tpu as pltpu
from jax.experimental.pallas import tpu_sc as plsc

B = 16384
L = 32
D_FEAT = 128
D_EDGE = 16
T_DIM = 128
T_LO = 64

NC = 2
NS = 16
NW = NC * NS
ROWS = B // NW
CSUB = 4
NSUB = ROWS // CSUB
CUR_CHUNK = 64
NCUR = ROWS // CUR_CHUNK


def _sc_body(node_tab, edge_tab, nbr_ids, eid_ids, cur_ids,
             node_sum, edge_sum, cur_rows,
             idx_n0, idx_n1, idx_e0, idx_e1,
             rows_n0, rows_n1, rows_e0, rows_e1,
             acc_n0, acc_n1, acc_e0, acc_e1,
             cidx0, cidx1, crows0, crows1,
             sem_i0, sem_i1, sem_g0, sem_g1, sem_w0, sem_w1,
             sem_c0, sem_c1, sem_cw0, sem_cw1):
    idx_n = (idx_n0, idx_n1)
    idx_e = (idx_e0, idx_e1)
    rows_n = (rows_n0, rows_n1)
    rows_e = (rows_e0, rows_e1)
    acc_n = (acc_n0, acc_n1)
    acc_e = (acc_e0, acc_e1)
    cidx = (cidx0, cidx1)
    crows = (crows0, crows1)
    sem_i = (sem_i0, sem_i1)
    sem_g = (sem_g0, sem_g1)
    sem_w = (sem_w0, sem_w1)
    sem_c = (sem_c0, sem_c1)
    sem_cw = (sem_cw0, sem_cw1)

    wid = lax.axis_index("s") * NC + lax.axis_index("c")
    base = wid * ROWS

    pltpu.sync_copy(nbr_ids.at[pl.ds(base * L, CSUB * L)], idx_n0)
    pltpu.sync_copy(eid_ids.at[pl.ds(base * L, CSUB * L)], idx_e0)
    pltpu.sync_copy(nbr_ids.at[pl.ds((base + CSUB) * L, CSUB * L)], idx_n1)
    pltpu.sync_copy(eid_ids.at[pl.ds((base + CSUB) * L, CSUB * L)], idx_e1)
    pltpu.async_copy(node_tab.at[idx_n0], rows_n0, sem_g0)
    pltpu.async_copy(edge_tab.at[idx_e0], rows_e0, sem_g0)

    def step_body(step, carry):
        for k in (0, 1):
            o = 1 - k
            s = step * 2 + k
            row0 = base + s * CSUB

            @pl.when(s == 0)
            def _():
                pltpu.async_copy(node_tab.at[idx_n[1]], rows_n[1], sem_g[1])
                pltpu.async_copy(edge_tab.at[idx_e[1]], rows_e[1], sem_g[1])

            @pl.when(jnp.logical_and(s >= 1, s < NSUB - 1))
            def _():
                pltpu.make_async_copy(
                    nbr_ids.at[pl.ds(0, CSUB * L)], idx_n[o], sem_i[o]).wait()
                pltpu.make_async_copy(
                    eid_ids.at[pl.ds(0, CSUB * L)], idx_e[o], sem_i[o]).wait()
                pltpu.async_copy(node_tab.at[idx_n[o]], rows_n[o], sem_g[o])
                pltpu.async_copy(edge_tab.at[idx_e[o]], rows_e[o], sem_g[o])

            pltpu.make_async_copy(
                node_tab.at[idx_n[k]], rows_n[k], sem_g[k]).wait()
            pltpu.make_async_copy(
                edge_tab.at[idx_e[k]], rows_e[k], sem_g[k]).wait()

            @pl.when(s < NSUB - 2)
            def _():
                nxt = (row0 + 2 * CSUB) * L
                pltpu.async_copy(
                    nbr_ids.at[pl.ds(nxt, CSUB * L)], idx_n[k], sem_i[k])
                pltpu.async_copy(
                    eid_ids.at[pl.ds(nxt, CSUB * L)], idx_e[k], sem_i[k])

            @pl.when(s >= 2)
            def _():
                pltpu.make_async_copy(
                    acc_n[k], node_sum.at[pl.ds(0, CSUB)], sem_w[k]).wait()
                pltpu.make_async_copy(
                    acc_e[k], edge_sum.at[pl.ds(0, CSUB)], sem_w[k]).wait()

            rn = rows_n[k]
            re = rows_e[k]
            an = acc_n[k]
            ae = acc_e[k]
            for bl in range(CSUB):
                r0 = bl * L
                for v in range(D_FEAT // 16):
                    col = pl.ds(v * 16, 16)
                    acc = rn[r0, col]
                    for l in range(1, L):
                        acc = acc + rn[r0 + l, col]
                    an[bl, col] = acc
                eacc = re[r0, :]
                for l in range(1, L):
                    eacc = eacc + re[r0 + l, :]
                ae[bl, :] = eacc

            pltpu.async_copy(an, node_sum.at[pl.ds(row0, CSUB)], sem_w[k])
            pltpu.async_copy(ae, edge_sum.at[pl.ds(row0, CSUB)], sem_w[k])
        return carry

    lax.fori_loop(0, NSUB // 2, step_body, 0)

    pltpu.make_async_copy(acc_n0, node_sum.at[pl.ds(0, CSUB)], sem_w0).wait()
    pltpu.make_async_copy(acc_e0, edge_sum.at[pl.ds(0, CSUB)], sem_w0).wait()
    pltpu.make_async_copy(acc_n1, node_sum.at[pl.ds(0, CSUB)], sem_w1).wait()
    pltpu.make_async_copy(acc_e1, edge_sum.at[pl.ds(0, CSUB)], sem_w1).wait()

    pltpu.sync_copy(cur_ids.at[pl.ds(base, CUR_CHUNK)], cidx0)
    pltpu.async_copy(node_tab.at[cidx0], crows0, sem_c0)

    def cur_body(i2, carry):
        for k in (0, 1):
            o = 1 - k
            c = i2 * 2 + k
            cb = base + c * CUR_CHUNK

            @pl.when(c < NCUR - 1)
            def _():
                pltpu.sync_copy(
                    cur_ids.at[pl.ds(cb + CUR_CHUNK, CUR_CHUNK)], cidx[o])

                @pl.when(c >= 1)
                def _():
                    pltpu.make_async_copy(
                        crows[o], cur_rows.at[pl.ds(0, CUR_CHUNK)],
                        sem_cw[o]).wait()

                pltpu.async_copy(node_tab.at[cidx[o]], crows[o], sem_c[o])

            pltpu.make_async_copy(
                node_tab.at[cidx[k]], crows[k], sem_c[k]).wait()
            pltpu.async_copy(
                crows[k], cur_rows.at[pl.ds(cb, CUR_CHUNK)], sem_cw[k])
        return carry

    lax.fori_loop(0, NCUR // 2, cur_body, 0)
    pltpu.make_async_copy(crows0, cur_rows.at[pl.ds(0, CUR_CHUNK)], sem_cw0).wait()
    pltpu.make_async_copy(crows1, cur_rows.at[pl.ds(0, CUR_CHUNK)], sem_cw1).wait()


@jax.jit
def _sc_gather(node_tab, edge_tab, nbr_ids, eid_ids, cur_ids):
    mesh = plsc.VectorSubcoreMesh(core_axis_name="c", subcore_axis_name="s")
    f = pl.kernel(
        _sc_body,
        out_type=[
            jax.ShapeDtypeStruct((B, D_FEAT), jnp.float32),
            jax.ShapeDtypeStruct((B, D_EDGE), jnp.float32),
            jax.ShapeDtypeStruct((B, D_FEAT), jnp.float32),
        ],
        mesh=mesh,
        scratch_types=[
            pltpu.VMEM((CSUB * L,), jnp.int32),
            pltpu.VMEM((CSUB * L,), jnp.int32),
            pltpu.VMEM((CSUB * L,), jnp.int32),
            pltpu.VMEM((CSUB * L,), jnp.int32),
            pltpu.VMEM((CSUB * L, D_FEAT), jnp.float32),
            pltpu.VMEM((CSUB * L, D_FEAT), jnp.float32),
            pltpu.VMEM((CSUB * L, D_EDGE), jnp.float32),
            pltpu.VMEM((CSUB * L, D_EDGE), jnp.float32),
            pltpu.VMEM((CSUB, D_FEAT), jnp.float32),
            pltpu.VMEM((CSUB, D_FEAT), jnp.float32),
            pltpu.VMEM((CSUB, D_EDGE), jnp.float32),
            pltpu.VMEM((CSUB, D_EDGE), jnp.float32),
            pltpu.VMEM((CUR_CHUNK,), jnp.int32),
            pltpu.VMEM((CUR_CHUNK,), jnp.int32),
            pltpu.VMEM((CUR_CHUNK, D_FEAT), jnp.float32),
            pltpu.VMEM((CUR_CHUNK, D_FEAT), jnp.float32),
            pltpu.SemaphoreType.DMA,
            pltpu.SemaphoreType.DMA,
            pltpu.SemaphoreType.DMA,
            pltpu.SemaphoreType.DMA,
            pltpu.SemaphoreType.DMA,
            pltpu.SemaphoreType.DMA,
            pltpu.SemaphoreType.DMA,
            pltpu.SemaphoreType.DMA,
            pltpu.SemaphoreType.DMA,
            pltpu.SemaphoreType.DMA,
        ],
        compiler_params=pltpu.CompilerParams(use_tc_tiling_on_sc=False),
    )
    return f(node_tab, edge_tab, nbr_ids, eid_ids, cur_ids)


_W_TIME = (1.0 / (10.0 ** np.linspace(0.0, 9.0, T_DIM))).astype(np.float32)
_WLO = (_W_TIME[:T_LO] / (2.0 * np.pi)).astype(np.float32)[None, :]
_WHI = _W_TIME[T_LO:].astype(np.float64)
_CM2 = (-(_WHI ** 2) / 2.0).astype(np.float32)[None, :]
_CM4 = ((_WHI ** 4) / 24.0).astype(np.float32)[None, :]
_CM6 = (-(_WHI ** 6) / 720.0).astype(np.float32)[None, :]

_MAGIC = np.float32(12582912.0)
_PC = (0.9999994436793985, -19.73903437293113, 64.9306133699045,
       -85.29597096153829, 58.91255532441485, -21.28302159300549)


def _poly_cos(u):
    kk = (u + _MAGIC) - _MAGIC
    f = u - kk
    s2 = f * f
    p = jnp.float32(_PC[5])
    p = p * s2 + jnp.float32(_PC[4])
    p = p * s2 + jnp.float32(_PC[3])
    p = p * s2 + jnp.float32(_PC[2])
    p = p * s2 + jnp.float32(_PC[1])
    p = p * s2 + jnp.float32(_PC[0])
    return p


TBT = 512


def _tc_time_body(t_ref, spt_ref, dpt_ref, wlo_ref, cm2_ref, cm4_ref, cm6_ref,
                  so_ref, do_ref):
    w = wlo_ref[...]

    def branch(pt):
        dt = t_ref[...] - pt
        d2 = dt * dt
        m2 = jnp.sum(d2, axis=1, keepdims=True)
        d4 = d2 * d2
        m4 = jnp.sum(d4, axis=1, keepdims=True)
        m6 = jnp.sum(d4 * d2, axis=1, keepdims=True)
        hi = (jnp.float32(L) + m2 * cm2_ref[...] + m4 * cm4_ref[...]
              + m6 * cm6_ref[...])
        lo = _poly_cos(dt[:, 0:1] * w)
        for l in range(1, L):
            lo = lo + _poly_cos(dt[:, l:l + 1] * w)
        return lo, hi

    lo, hi = branch(spt_ref[...])
    so_ref[:, 0:T_LO] = lo
    so_ref[:, T_LO:T_DIM] = hi
    lo, hi = branch(dpt_ref[...])
    do_ref[:, 0:T_LO] = lo
    do_ref[:, T_LO:T_DIM] = hi


@jax.jit
def _tc_time(t, spt, dpt, wlo, cm2, cm4, cm6):
    grid = (B // TBT,)
    return pl.pallas_call(
        _tc_time_body,
        grid=grid,
        in_specs=[
            pl.BlockSpec((TBT, 1), lambda i: (i, 0)),
            pl.BlockSpec((TBT, L), lambda i: (i, 0)),
            pl.BlockSpec((TBT, L), lambda i: (i, 0)),
            pl.BlockSpec((1, T_LO), lambda i: (0, 0)),
            pl.BlockSpec((1, T_DIM - T_LO), lambda i: (0, 0)),
            pl.BlockSpec((1, T_DIM - T_LO), lambda i: (0, 0)),
            pl.BlockSpec((1, T_DIM - T_LO), lambda i: (0, 0)),
        ],
        out_specs=[
            pl.BlockSpec((TBT, T_DIM), lambda i: (i, 0)),
            pl.BlockSpec((TBT, T_DIM), lambda i: (i, 0)),
        ],
        out_shape=[
            jax.ShapeDtypeStruct((B, T_DIM), jnp.float32),
            jax.ShapeDtypeStruct((B, T_DIM), jnp.float32),
        ],
    )(t, spt, dpt, wlo, cm2, cm4, cm6)


TBM = 1024


def _tc_mm_body(scr_ref, sns_ref, ses_ref, sts_ref,
                dcr_ref, dns_ref, des_ref, dts_ref,
                wc_ref, wn_ref, we_ref, wt_ref, b_ref,
                so_ref, do_ref):
    def branch(cr, ns, es, ts):
        acc = jnp.dot(cr, wc_ref[...], preferred_element_type=jnp.float32)
        acc = acc + jnp.dot(ns, wn_ref[...], preferred_element_type=jnp.float32)
        acc = acc + jnp.dot(es, we_ref[...], preferred_element_type=jnp.float32)
        acc = acc + jnp.dot(ts, wt_ref[...], preferred_element_type=jnp.float32)
        return jnp.maximum(acc + b_ref[...], 0.0)

    so_ref[...] = branch(scr_ref[...], sns_ref[...], ses_ref[...], sts_ref[...])
    do_ref[...] = branch(dcr_ref[...], dns_ref[...], des_ref[...], dts_ref[...])


@jax.jit
def _tc_mm(scr, sns, ses, sts, dcr, dns, des, dts, WcT, WnT, WeT, WtT, bias):
    grid = (B // TBM,)
    row = lambda i: (i, 0)
    fix = lambda i: (0, 0)
    return pl.pallas_call(
        _tc_mm_body,
        grid=grid,
        in_specs=[
            pl.BlockSpec((TBM, D_FEAT), row),
            pl.BlockSpec((TBM, D_FEAT), row),
            pl.BlockSpec((TBM, D_EDGE), row),
            pl.BlockSpec((TBM, T_DIM), row),
            pl.BlockSpec((TBM, D_FEAT), row),
            pl.BlockSpec((TBM, D_FEAT), row),
            pl.BlockSpec((TBM, D_EDGE), row),
            pl.BlockSpec((TBM, T_DIM), row),
            pl.BlockSpec((D_FEAT, D_FEAT), fix),
            pl.BlockSpec((D_FEAT, D_FEAT), fix),
            pl.BlockSpec((D_EDGE, D_FEAT), fix),
            pl.BlockSpec((T_DIM, D_FEAT), fix),
            pl.BlockSpec((1, D_FEAT), fix),
        ],
        out_specs=[
            pl.BlockSpec((TBM, D_FEAT), row),
            pl.BlockSpec((TBM, D_FEAT), row),
        ],
        out_shape=[
            jax.ShapeDtypeStruct((B, D_FEAT), jnp.float32),
            jax.ShapeDtypeStruct((B, D_FEAT), jnp.float32),
        ],
    )(scr, sns, ses, sts, dcr, dns, des, dts, WcT, WnT, WeT, WtT, bias)


def kernel(src_node_ids, dst_node_ids, node_interact_times,
           src_padded_nodes_neighbor_ids, src_padded_nodes_edge_ids, src_padded_nodes_neighbor_times,
           dst_padded_nodes_neighbor_ids, dst_padded_nodes_edge_ids, dst_padded_nodes_neighbor_times,
           node_raw_features, edge_raw_features, W, b):
    s_nbr = src_padded_nodes_neighbor_ids.reshape(-1).astype(jnp.int32)
    s_eid = src_padded_nodes_edge_ids.reshape(-1).astype(jnp.int32)
    d_nbr = dst_padded_nodes_neighbor_ids.reshape(-1).astype(jnp.int32)
    d_eid = dst_padded_nodes_edge_ids.reshape(-1).astype(jnp.int32)
    s_cur = src_node_ids.astype(jnp.int32)
    d_cur = dst_node_ids.astype(jnp.int32)
    t = node_interact_times[:, None]

    sns, ses, scr = _sc_gather(node_raw_features, edge_raw_features,
                               s_nbr, s_eid, s_cur)
    dns, des, dcr = _sc_gather(node_raw_features, edge_raw_features,
                               d_nbr, d_eid, d_cur)

    sts, dts = _tc_time(t, src_padded_nodes_neighbor_times,
                        dst_padded_nodes_neighbor_times,
                        jnp.asarray(_WLO), jnp.asarray(_CM2),
                        jnp.asarray(_CM4), jnp.asarray(_CM6))

    inv_l = jnp.float32(1.0 / L)
    WcT = W[:, :D_FEAT].T
    WnT = (W[:, D_FEAT:2 * D_FEAT] * inv_l).T
    WeT = (W[:, 2 * D_FEAT:2 * D_FEAT + D_EDGE] * inv_l).T
    WtT = (W[:, 2 * D_FEAT + D_EDGE:] * inv_l).T
    bias = b[None, :]

    out_s, out_d = _tc_mm(scr, sns, ses, sts, dcr, dns, des, dts,
                          WcT, WnT, WeT, WtT, bias)
    return (out_s, out_d, jnp.zeros_like(out_s))


# speedup vs baseline: 7.2966x; 1.3994x over previous
import functools

import numpy as np
import jax
import jax.numpy as jnp
from jax import lax
from jax.experimental import pallas as pl
from jax.experimental.pallas import tpu as pltpu
from jax.experimental.pallas import tpu_sc as plsc

B = 16384
L = 32
D_FEAT = 128
D_EDGE = 16
T_DIM = 128
T_LO = 64

NC = 2
NS = 16
NW = NC * NS
ROWS = B // NW
CSUB = 4
NSUB = ROWS // CSUB
CUR_CHUNK = 64
NCUR = ROWS // CUR_CHUNK


def _sc_body(node_tab, edge_tab, nbr_ids, eid_ids, cur_ids,
             node_sum, edge_sum, cur_rows,
             idx_n0, idx_n1, idx_e0, idx_e1,
             rows_n0, rows_n1, rows_e0, rows_e1,
             acc_n0, acc_n1, acc_e0, acc_e1,
             cidx0, cidx1, crows0, crows1,
             sem_i0, sem_i1, sem_g0, sem_g1, sem_w0, sem_w1,
             sem_c0, sem_c1, sem_cw0, sem_cw1):
    idx_n = (idx_n0, idx_n1)
    idx_e = (idx_e0, idx_e1)
    rows_n = (rows_n0, rows_n1)
    rows_e = (rows_e0, rows_e1)
    acc_n = (acc_n0, acc_n1)
    acc_e = (acc_e0, acc_e1)
    cidx = (cidx0, cidx1)
    crows = (crows0, crows1)
    sem_i = (sem_i0, sem_i1)
    sem_g = (sem_g0, sem_g1)
    sem_w = (sem_w0, sem_w1)
    sem_c = (sem_c0, sem_c1)
    sem_cw = (sem_cw0, sem_cw1)

    wid = lax.axis_index("s") * NC + lax.axis_index("c")
    base = wid * ROWS

    pltpu.sync_copy(nbr_ids.at[pl.ds(base * L, CSUB * L)], idx_n0)
    pltpu.sync_copy(eid_ids.at[pl.ds(base * L, CSUB * L)], idx_e0)
    pltpu.sync_copy(nbr_ids.at[pl.ds((base + CSUB) * L, CSUB * L)], idx_n1)
    pltpu.sync_copy(eid_ids.at[pl.ds((base + CSUB) * L, CSUB * L)], idx_e1)
    pltpu.async_copy(node_tab.at[idx_n0], rows_n0, sem_g0)
    pltpu.async_copy(edge_tab.at[idx_e0], rows_e0, sem_g0)

    def step_body(step, carry):
        for k in (0, 1):
            o = 1 - k
            s = step * 2 + k
            row0 = base + s * CSUB

            @pl.when(s == 0)
            def _():
                pltpu.async_copy(node_tab.at[idx_n[1]], rows_n[1], sem_g[1])
                pltpu.async_copy(edge_tab.at[idx_e[1]], rows_e[1], sem_g[1])

            @pl.when(jnp.logical_and(s >= 1, s < NSUB - 1))
            def _():
                pltpu.make_async_copy(
                    nbr_ids.at[pl.ds(0, CSUB * L)], idx_n[o], sem_i[o]).wait()
                pltpu.make_async_copy(
                    eid_ids.at[pl.ds(0, CSUB * L)], idx_e[o], sem_i[o]).wait()
                pltpu.async_copy(node_tab.at[idx_n[o]], rows_n[o], sem_g[o])
                pltpu.async_copy(edge_tab.at[idx_e[o]], rows_e[o], sem_g[o])

            pltpu.make_async_copy(
                node_tab.at[idx_n[k]], rows_n[k], sem_g[k]).wait()
            pltpu.make_async_copy(
                edge_tab.at[idx_e[k]], rows_e[k], sem_g[k]).wait()

            @pl.when(s < NSUB - 2)
            def _():
                nxt = (row0 + 2 * CSUB) * L
                pltpu.async_copy(
                    nbr_ids.at[pl.ds(nxt, CSUB * L)], idx_n[k], sem_i[k])
                pltpu.async_copy(
                    eid_ids.at[pl.ds(nxt, CSUB * L)], idx_e[k], sem_i[k])

            @pl.when(s >= 2)
            def _():
                pltpu.make_async_copy(
                    acc_n[k], node_sum.at[pl.ds(0, CSUB)], sem_w[k]).wait()
                pltpu.make_async_copy(
                    acc_e[k], edge_sum.at[pl.ds(0, CSUB)], sem_w[k]).wait()

            rn = rows_n[k]
            re = rows_e[k]
            an = acc_n[k]
            ae = acc_e[k]
            for bl in range(CSUB):
                r0 = bl * L

                def vbody(v, c2, rn=rn, an=an, r0=r0, bl=bl):
                    col = pl.ds(pl.multiple_of(v * 16, 16), 16)
                    parts = [rn[r0 + p, col] for p in range(8)]
                    for l in range(8, L):
                        parts[l % 8] = parts[l % 8] + rn[r0 + l, col]
                    acc = (((parts[0] + parts[1]) + (parts[2] + parts[3]))
                           + ((parts[4] + parts[5]) + (parts[6] + parts[7])))
                    an[bl, col] = acc
                    return c2

                lax.fori_loop(0, D_FEAT // 16, vbody, 0)
                eparts = [re[r0 + p, :] for p in range(8)]
                for l in range(8, L):
                    eparts[l % 8] = eparts[l % 8] + re[r0 + l, :]
                ae[bl, :] = (((eparts[0] + eparts[1]) + (eparts[2] + eparts[3]))
                             + ((eparts[4] + eparts[5]) + (eparts[6] + eparts[7])))

            pltpu.async_copy(an, node_sum.at[pl.ds(row0, CSUB)], sem_w[k])
            pltpu.async_copy(ae, edge_sum.at[pl.ds(row0, CSUB)], sem_w[k])
        return carry

    lax.fori_loop(0, NSUB // 2, step_body, 0)

    pltpu.make_async_copy(acc_n0, node_sum.at[pl.ds(0, CSUB)], sem_w0).wait()
    pltpu.make_async_copy(acc_e0, edge_sum.at[pl.ds(0, CSUB)], sem_w0).wait()
    pltpu.make_async_copy(acc_n1, node_sum.at[pl.ds(0, CSUB)], sem_w1).wait()
    pltpu.make_async_copy(acc_e1, edge_sum.at[pl.ds(0, CSUB)], sem_w1).wait()

    pltpu.sync_copy(cur_ids.at[pl.ds(base, CUR_CHUNK)], cidx0)
    pltpu.async_copy(node_tab.at[cidx0], crows0, sem_c0)

    def cur_body(i2, carry):
        for k in (0, 1):
            o = 1 - k
            c = i2 * 2 + k
            cb = base + c * CUR_CHUNK

            @pl.when(c < NCUR - 1)
            def _():
                pltpu.sync_copy(
                    cur_ids.at[pl.ds(cb + CUR_CHUNK, CUR_CHUNK)], cidx[o])

                @pl.when(c >= 1)
                def _():
                    pltpu.make_async_copy(
                        crows[o], cur_rows.at[pl.ds(0, CUR_CHUNK)],
                        sem_cw[o]).wait()

                pltpu.async_copy(node_tab.at[cidx[o]], crows[o], sem_c[o])

            pltpu.make_async_copy(
                node_tab.at[cidx[k]], crows[k], sem_c[k]).wait()
            pltpu.async_copy(
                crows[k], cur_rows.at[pl.ds(cb, CUR_CHUNK)], sem_cw[k])
        return carry

    lax.fori_loop(0, NCUR // 2, cur_body, 0)
    pltpu.make_async_copy(crows0, cur_rows.at[pl.ds(0, CUR_CHUNK)], sem_cw0).wait()
    pltpu.make_async_copy(crows1, cur_rows.at[pl.ds(0, CUR_CHUNK)], sem_cw1).wait()


@jax.jit
def _sc_gather(node_tab, edge_tab, nbr_ids, eid_ids, cur_ids):
    mesh = plsc.VectorSubcoreMesh(core_axis_name="c", subcore_axis_name="s")
    f = pl.kernel(
        _sc_body,
        out_type=[
            jax.ShapeDtypeStruct((B, D_FEAT), jnp.float32),
            jax.ShapeDtypeStruct((B, D_EDGE), jnp.float32),
            jax.ShapeDtypeStruct((B, D_FEAT), jnp.float32),
        ],
        mesh=mesh,
        scratch_types=[
            pltpu.VMEM((CSUB * L,), jnp.int32),
            pltpu.VMEM((CSUB * L,), jnp.int32),
            pltpu.VMEM((CSUB * L,), jnp.int32),
            pltpu.VMEM((CSUB * L,), jnp.int32),
            pltpu.VMEM((CSUB * L, D_FEAT), jnp.float32),
            pltpu.VMEM((CSUB * L, D_FEAT), jnp.float32),
            pltpu.VMEM((CSUB * L, D_EDGE), jnp.float32),
            pltpu.VMEM((CSUB * L, D_EDGE), jnp.float32),
            pltpu.VMEM((CSUB, D_FEAT), jnp.float32),
            pltpu.VMEM((CSUB, D_FEAT), jnp.float32),
            pltpu.VMEM((CSUB, D_EDGE), jnp.float32),
            pltpu.VMEM((CSUB, D_EDGE), jnp.float32),
            pltpu.VMEM((CUR_CHUNK,), jnp.int32),
            pltpu.VMEM((CUR_CHUNK,), jnp.int32),
            pltpu.VMEM((CUR_CHUNK, D_FEAT), jnp.float32),
            pltpu.VMEM((CUR_CHUNK, D_FEAT), jnp.float32),
            pltpu.SemaphoreType.DMA,
            pltpu.SemaphoreType.DMA,
            pltpu.SemaphoreType.DMA,
            pltpu.SemaphoreType.DMA,
            pltpu.SemaphoreType.DMA,
            pltpu.SemaphoreType.DMA,
            pltpu.SemaphoreType.DMA,
            pltpu.SemaphoreType.DMA,
            pltpu.SemaphoreType.DMA,
            pltpu.SemaphoreType.DMA,
        ],
        compiler_params=pltpu.CompilerParams(use_tc_tiling_on_sc=False),
    )
    return f(node_tab, edge_tab, nbr_ids, eid_ids, cur_ids)


_W_TIME = (1.0 / (10.0 ** np.linspace(0.0, 9.0, T_DIM))).astype(np.float32)
_WLO = (_W_TIME[:T_LO] / (2.0 * np.pi)).astype(np.float32)[None, :]
_WHI = _W_TIME[T_LO:].astype(np.float64)
_CM2 = (-(_WHI ** 2) / 2.0).astype(np.float32)[None, :]
_CM4 = ((_WHI ** 4) / 24.0).astype(np.float32)[None, :]
_CM6 = (-(_WHI ** 6) / 720.0).astype(np.float32)[None, :]

_MAGIC = np.float32(12582912.0)
_PC = (0.9999994436793985, -19.73903437293113, 64.9306133699045,
       -85.29597096153829, 58.91255532441485, -21.28302159300549)


def _poly_cos(u):
    kk = (u + _MAGIC) - _MAGIC
    f = u - kk
    s2 = f * f
    p = jnp.float32(_PC[5])
    p = p * s2 + jnp.float32(_PC[4])
    p = p * s2 + jnp.float32(_PC[3])
    p = p * s2 + jnp.float32(_PC[2])
    p = p * s2 + jnp.float32(_PC[1])
    p = p * s2 + jnp.float32(_PC[0])
    return p


TBT = 512


def _tc_time_body(t_ref, spt_ref, dpt_ref, wlo_ref, cm2_ref, cm4_ref, cm6_ref,
                  so_ref, do_ref):
    w = wlo_ref[...]

    def branch(pt):
        dt = t_ref[...] - pt
        d2 = dt * dt
        m2 = jnp.sum(d2, axis=1, keepdims=True)
        d4 = d2 * d2
        m4 = jnp.sum(d4, axis=1, keepdims=True)
        m6 = jnp.sum(d4 * d2, axis=1, keepdims=True)
        hi = (jnp.float32(L) + m2 * cm2_ref[...] + m4 * cm4_ref[...]
              + m6 * cm6_ref[...])
        lo = _poly_cos(dt[:, 0:1] * w)
        for l in range(1, L):
            lo = lo + _poly_cos(dt[:, l:l + 1] * w)
        return lo, hi

    lo, hi = branch(spt_ref[...])
    so_ref[:, 0:T_LO] = lo
    so_ref[:, T_LO:T_DIM] = hi
    lo, hi = branch(dpt_ref[...])
    do_ref[:, 0:T_LO] = lo
    do_ref[:, T_LO:T_DIM] = hi


@jax.jit
def _tc_time(t, spt, dpt, wlo, cm2, cm4, cm6):
    grid = (B // TBT,)
    return pl.pallas_call(
        _tc_time_body,
        grid=grid,
        in_specs=[
            pl.BlockSpec((TBT, 1), lambda i: (i, 0)),
            pl.BlockSpec((TBT, L), lambda i: (i, 0)),
            pl.BlockSpec((TBT, L), lambda i: (i, 0)),
            pl.BlockSpec((1, T_LO), lambda i: (0, 0)),
            pl.BlockSpec((1, T_DIM - T_LO), lambda i: (0, 0)),
            pl.BlockSpec((1, T_DIM - T_LO), lambda i: (0, 0)),
            pl.BlockSpec((1, T_DIM - T_LO), lambda i: (0, 0)),
        ],
        out_specs=[
            pl.BlockSpec((TBT, T_DIM), lambda i: (i, 0)),
            pl.BlockSpec((TBT, T_DIM), lambda i: (i, 0)),
        ],
        out_shape=[
            jax.ShapeDtypeStruct((B, T_DIM), jnp.float32),
            jax.ShapeDtypeStruct((B, T_DIM), jnp.float32),
        ],
    )(t, spt, dpt, wlo, cm2, cm4, cm6)


TBM = 1024


def _tc_mm_body(scr_ref, sns_ref, ses_ref, sts_ref,
                dcr_ref, dns_ref, des_ref, dts_ref,
                wc_ref, wn_ref, we_ref, wt_ref, b_ref,
                so_ref, do_ref):
    def branch(cr, ns, es, ts):
        acc = jnp.dot(cr, wc_ref[...], preferred_element_type=jnp.float32)
        acc = acc + jnp.dot(ns, wn_ref[...], preferred_element_type=jnp.float32)
        acc = acc + jnp.dot(es, we_ref[...], preferred_element_type=jnp.float32)
        acc = acc + jnp.dot(ts, wt_ref[...], preferred_element_type=jnp.float32)
        return jnp.maximum(acc + b_ref[...], 0.0)

    so_ref[...] = branch(scr_ref[...], sns_ref[...], ses_ref[...], sts_ref[...])
    do_ref[...] = branch(dcr_ref[...], dns_ref[...], des_ref[...], dts_ref[...])


@jax.jit
def _tc_mm(scr, sns, ses, sts, dcr, dns, des, dts, WcT, WnT, WeT, WtT, bias):
    grid = (B // TBM,)
    row = lambda i: (i, 0)
    fix = lambda i: (0, 0)
    return pl.pallas_call(
        _tc_mm_body,
        grid=grid,
        in_specs=[
            pl.BlockSpec((TBM, D_FEAT), row),
            pl.BlockSpec((TBM, D_FEAT), row),
            pl.BlockSpec((TBM, D_EDGE), row),
            pl.BlockSpec((TBM, T_DIM), row),
            pl.BlockSpec((TBM, D_FEAT), row),
            pl.BlockSpec((TBM, D_FEAT), row),
            pl.BlockSpec((TBM, D_EDGE), row),
            pl.BlockSpec((TBM, T_DIM), row),
            pl.BlockSpec((D_FEAT, D_FEAT), fix),
            pl.BlockSpec((D_FEAT, D_FEAT), fix),
            pl.BlockSpec((D_EDGE, D_FEAT), fix),
            pl.BlockSpec((T_DIM, D_FEAT), fix),
            pl.BlockSpec((1, D_FEAT), fix),
        ],
        out_specs=[
            pl.BlockSpec((TBM, D_FEAT), row),
            pl.BlockSpec((TBM, D_FEAT), row),
        ],
        out_shape=[
            jax.ShapeDtypeStruct((B, D_FEAT), jnp.float32),
            jax.ShapeDtypeStruct((B, D_FEAT), jnp.float32),
        ],
    )(scr, sns, ses, sts, dcr, dns, des, dts, WcT, WnT, WeT, WtT, bias)


def kernel(src_node_ids, dst_node_ids, node_interact_times,
           src_padded_nodes_neighbor_ids, src_padded_nodes_edge_ids, src_padded_nodes_neighbor_times,
           dst_padded_nodes_neighbor_ids, dst_padded_nodes_edge_ids, dst_padded_nodes_neighbor_times,
           node_raw_features, edge_raw_features, W, b):
    s_nbr = src_padded_nodes_neighbor_ids.reshape(-1).astype(jnp.int32)
    s_eid = src_padded_nodes_edge_ids.reshape(-1).astype(jnp.int32)
    d_nbr = dst_padded_nodes_neighbor_ids.reshape(-1).astype(jnp.int32)
    d_eid = dst_padded_nodes_edge_ids.reshape(-1).astype(jnp.int32)
    s_cur = src_node_ids.astype(jnp.int32)
    d_cur = dst_node_ids.astype(jnp.int32)
    t = node_interact_times[:, None]

    sns, ses, scr = _sc_gather(node_raw_features, edge_raw_features,
                               s_nbr, s_eid, s_cur)
    dns, des, dcr = _sc_gather(node_raw_features, edge_raw_features,
                               d_nbr, d_eid, d_cur)

    sts, dts = _tc_time(t, src_padded_nodes_neighbor_times,
                        dst_padded_nodes_neighbor_times,
                        jnp.asarray(_WLO), jnp.asarray(_CM2),
                        jnp.asarray(_CM4), jnp.asarray(_CM6))

    inv_l = jnp.float32(1.0 / L)
    WcT = W[:, :D_FEAT].T
    WnT = (W[:, D_FEAT:2 * D_FEAT] * inv_l).T
    WeT = (W[:, 2 * D_FEAT:2 * D_FEAT + D_EDGE] * inv_l).T
    WtT = (W[:, 2 * D_FEAT + D_EDGE:] * inv_l).T
    bias = b[None, :]

    out_s, out_d = _tc_mm(scr, sns, ses, sts, dcr, dns, des, dts,
                          WcT, WnT, WeT, WtT, bias)
    return (out_s, out_d, jnp.zeros_like(out_s))


# speedup vs baseline: 7.3760x; 1.0109x over previous
import functools

import numpy as np
import jax
import jax.numpy as jnp
from jax import lax
from jax.experimental import pallas as pl
from jax.experimental.pallas import tpu as pltpu
from jax.experimental.pallas import tpu_sc as plsc

B = 16384
L = 32
D_FEAT = 128
D_EDGE = 16
T_DIM = 128
T_LO = 64

NC = 2
NS = 16
NW = NC * NS
ROWS = B // NW
CSUB = 4
NSUB = ROWS // CSUB
CUR_CHUNK = 64
NCUR = ROWS // CUR_CHUNK


def _sc_body(node_tab, edge_tab, nbr_ids, eid_ids, cur_ids,
             node_sum, edge_sum, cur_rows,
             idx_n0, idx_n1, idx_e0, idx_e1,
             rows_n0, rows_n1, rows_e0, rows_e1,
             acc_n0, acc_n1, acc_e0, acc_e1,
             cidx0, cidx1, crows0, crows1,
             sem_i0, sem_i1, sem_g0, sem_g1, sem_w0, sem_w1,
             sem_c0, sem_c1, sem_cw0, sem_cw1):
    idx_n = (idx_n0, idx_n1)
    idx_e = (idx_e0, idx_e1)
    rows_n = (rows_n0, rows_n1)
    rows_e = (rows_e0, rows_e1)
    acc_n = (acc_n0, acc_n1)
    acc_e = (acc_e0, acc_e1)
    cidx = (cidx0, cidx1)
    crows = (crows0, crows1)
    sem_i = (sem_i0, sem_i1)
    sem_g = (sem_g0, sem_g1)
    sem_w = (sem_w0, sem_w1)
    sem_c = (sem_c0, sem_c1)
    sem_cw = (sem_cw0, sem_cw1)

    wid = lax.axis_index("s") * NC + lax.axis_index("c")
    base = wid * ROWS

    zv = jnp.zeros((16,), jnp.float32)
    for buf in (acc_e0, acc_e1):
        for bl in range(CSUB):
            for v in range(1, D_FEAT // 16):
                buf[bl, pl.ds(v * 16, 16)] = zv

    pltpu.sync_copy(nbr_ids.at[pl.ds(base * L, CSUB * L)], idx_n0)
    pltpu.sync_copy(eid_ids.at[pl.ds(base * L, CSUB * L)], idx_e0)
    pltpu.sync_copy(nbr_ids.at[pl.ds((base + CSUB) * L, CSUB * L)], idx_n1)
    pltpu.sync_copy(eid_ids.at[pl.ds((base + CSUB) * L, CSUB * L)], idx_e1)
    pltpu.async_copy(node_tab.at[idx_n0], rows_n0, sem_g0)
    pltpu.async_copy(edge_tab.at[idx_e0], rows_e0, sem_g0)

    def step_body(step, carry):
        for k in (0, 1):
            o = 1 - k
            s = step * 2 + k
            row0 = base + s * CSUB

            @pl.when(s == 0)
            def _():
                pltpu.async_copy(node_tab.at[idx_n[1]], rows_n[1], sem_g[1])
                pltpu.async_copy(edge_tab.at[idx_e[1]], rows_e[1], sem_g[1])

            @pl.when(jnp.logical_and(s >= 1, s < NSUB - 1))
            def _():
                pltpu.make_async_copy(
                    nbr_ids.at[pl.ds(0, CSUB * L)], idx_n[o], sem_i[o]).wait()
                pltpu.make_async_copy(
                    eid_ids.at[pl.ds(0, CSUB * L)], idx_e[o], sem_i[o]).wait()
                pltpu.async_copy(node_tab.at[idx_n[o]], rows_n[o], sem_g[o])
                pltpu.async_copy(edge_tab.at[idx_e[o]], rows_e[o], sem_g[o])

            pltpu.make_async_copy(
                node_tab.at[idx_n[k]], rows_n[k], sem_g[k]).wait()
            pltpu.make_async_copy(
                edge_tab.at[idx_e[k]], rows_e[k], sem_g[k]).wait()

            @pl.when(s < NSUB - 2)
            def _():
                nxt = (row0 + 2 * CSUB) * L
                pltpu.async_copy(
                    nbr_ids.at[pl.ds(nxt, CSUB * L)], idx_n[k], sem_i[k])
                pltpu.async_copy(
                    eid_ids.at[pl.ds(nxt, CSUB * L)], idx_e[k], sem_i[k])

            @pl.when(s >= 2)
            def _():
                pltpu.make_async_copy(
                    acc_n[k], node_sum.at[0, pl.ds(0, CSUB), :], sem_w[k]).wait()
                pltpu.make_async_copy(
                    acc_e[k], edge_sum.at[0, pl.ds(0, CSUB), :], sem_w[k]).wait()

            rn = rows_n[k]
            re = rows_e[k]
            an = acc_n[k]
            ae = acc_e[k]
            for bl in range(CSUB):
                r0 = bl * L

                def vbody(v, c2, rn=rn, an=an, r0=r0, bl=bl):
                    col = pl.ds(pl.multiple_of(v * 16, 16), 16)
                    parts = [rn[r0 + p, col] for p in range(8)]
                    for l in range(8, L):
                        parts[l % 8] = parts[l % 8] + rn[r0 + l, col]
                    acc = (((parts[0] + parts[1]) + (parts[2] + parts[3]))
                           + ((parts[4] + parts[5]) + (parts[6] + parts[7])))
                    an[bl, col] = acc
                    return c2

                lax.fori_loop(0, D_FEAT // 16, vbody, 0)
                eparts = [re[r0 + p, :] for p in range(8)]
                for l in range(8, L):
                    eparts[l % 8] = eparts[l % 8] + re[r0 + l, :]
                ae[bl, pl.ds(0, 16)] = (
                    ((eparts[0] + eparts[1]) + (eparts[2] + eparts[3]))
                    + ((eparts[4] + eparts[5]) + (eparts[6] + eparts[7])))

            blk = row0 // 8
            off = row0 % 8
            pltpu.async_copy(an, node_sum.at[blk, pl.ds(off, CSUB), :], sem_w[k])
            pltpu.async_copy(ae, edge_sum.at[blk, pl.ds(off, CSUB), :], sem_w[k])
        return carry

    lax.fori_loop(0, NSUB // 2, step_body, 0)

    pltpu.make_async_copy(acc_n0, node_sum.at[0, pl.ds(0, CSUB), :], sem_w0).wait()
    pltpu.make_async_copy(acc_e0, edge_sum.at[0, pl.ds(0, CSUB), :], sem_w0).wait()
    pltpu.make_async_copy(acc_n1, node_sum.at[0, pl.ds(0, CSUB), :], sem_w1).wait()
    pltpu.make_async_copy(acc_e1, edge_sum.at[0, pl.ds(0, CSUB), :], sem_w1).wait()

    pltpu.sync_copy(cur_ids.at[pl.ds(base, CUR_CHUNK)], cidx0)
    pltpu.async_copy(node_tab.at[cidx0], crows0, sem_c0)

    def cur_body(i2, carry):
        for k in (0, 1):
            o = 1 - k
            c = i2 * 2 + k
            cb = base + c * CUR_CHUNK

            @pl.when(c < NCUR - 1)
            def _():
                pltpu.sync_copy(
                    cur_ids.at[pl.ds(cb + CUR_CHUNK, CUR_CHUNK)], cidx[o])

                @pl.when(c >= 1)
                def _():
                    for j in range(CUR_CHUNK // 8):
                        pltpu.make_async_copy(
                            crows[o].at[pl.ds(j * 8, 8), :],
                            cur_rows.at[0], sem_cw[o]).wait()

                pltpu.async_copy(node_tab.at[cidx[o]], crows[o], sem_c[o])

            pltpu.make_async_copy(
                node_tab.at[cidx[k]], crows[k], sem_c[k]).wait()
            for j in range(CUR_CHUNK // 8):
                pltpu.async_copy(crows[k].at[pl.ds(j * 8, 8), :],
                                 cur_rows.at[cb // 8 + j], sem_cw[k])
        return carry

    lax.fori_loop(0, NCUR // 2, cur_body, 0)
    for j in range(CUR_CHUNK // 8):
        pltpu.make_async_copy(crows0.at[pl.ds(j * 8, 8), :],
                              cur_rows.at[0], sem_cw0).wait()
        pltpu.make_async_copy(crows1.at[pl.ds(j * 8, 8), :],
                              cur_rows.at[0], sem_cw1).wait()


@jax.jit
def _sc_gather(node_tab, edge_tab, nbr_ids, eid_ids, cur_ids):
    mesh = plsc.VectorSubcoreMesh(core_axis_name="c", subcore_axis_name="s")
    f = pl.kernel(
        _sc_body,
        out_type=[
            jax.ShapeDtypeStruct((B // 8, 8, D_FEAT), jnp.float32),
            jax.ShapeDtypeStruct((B // 8, 8, D_FEAT), jnp.float32),
            jax.ShapeDtypeStruct((B // 8, 8, D_FEAT), jnp.float32),
        ],
        mesh=mesh,
        scratch_types=[
            pltpu.VMEM((CSUB * L,), jnp.int32),
            pltpu.VMEM((CSUB * L,), jnp.int32),
            pltpu.VMEM((CSUB * L,), jnp.int32),
            pltpu.VMEM((CSUB * L,), jnp.int32),
            pltpu.VMEM((CSUB * L, D_FEAT), jnp.float32),
            pltpu.VMEM((CSUB * L, D_FEAT), jnp.float32),
            pltpu.VMEM((CSUB * L, D_EDGE), jnp.float32),
            pltpu.VMEM((CSUB * L, D_EDGE), jnp.float32),
            pltpu.VMEM((CSUB, D_FEAT), jnp.float32),
            pltpu.VMEM((CSUB, D_FEAT), jnp.float32),
            pltpu.VMEM((CSUB, D_FEAT), jnp.float32),
            pltpu.VMEM((CSUB, D_FEAT), jnp.float32),
            pltpu.VMEM((CUR_CHUNK,), jnp.int32),
            pltpu.VMEM((CUR_CHUNK,), jnp.int32),
            pltpu.VMEM((CUR_CHUNK, D_FEAT), jnp.float32),
            pltpu.VMEM((CUR_CHUNK, D_FEAT), jnp.float32),
            pltpu.SemaphoreType.DMA,
            pltpu.SemaphoreType.DMA,
            pltpu.SemaphoreType.DMA,
            pltpu.SemaphoreType.DMA,
            pltpu.SemaphoreType.DMA,
            pltpu.SemaphoreType.DMA,
            pltpu.SemaphoreType.DMA,
            pltpu.SemaphoreType.DMA,
            pltpu.SemaphoreType.DMA,
            pltpu.SemaphoreType.DMA,
        ],
        compiler_params=pltpu.CompilerParams(use_tc_tiling_on_sc=False),
    )
    return f(node_tab, edge_tab, nbr_ids, eid_ids, cur_ids)


_W_TIME = (1.0 / (10.0 ** np.linspace(0.0, 9.0, T_DIM))).astype(np.float32)
_WLO = (_W_TIME[:T_LO] / (2.0 * np.pi)).astype(np.float32)[None, :]
_WHI = _W_TIME[T_LO:].astype(np.float64)
_CM2 = (-(_WHI ** 2) / 2.0).astype(np.float32)[None, :]
_CM4 = ((_WHI ** 4) / 24.0).astype(np.float32)[None, :]
_CM6 = (-(_WHI ** 6) / 720.0).astype(np.float32)[None, :]

_MAGIC = np.float32(12582912.0)
_PC = (0.9999994436793985, -19.73903437293113, 64.9306133699045,
       -85.29597096153829, 58.91255532441485, -21.28302159300549)


def _poly_cos(u):
    kk = (u + _MAGIC) - _MAGIC
    f = u - kk
    s2 = f * f
    p = jnp.float32(_PC[5])
    p = p * s2 + jnp.float32(_PC[4])
    p = p * s2 + jnp.float32(_PC[3])
    p = p * s2 + jnp.float32(_PC[2])
    p = p * s2 + jnp.float32(_PC[1])
    p = p * s2 + jnp.float32(_PC[0])
    return p


TBT = 512


def _tc_time_body(t_ref, spt_ref, dpt_ref, wlo_ref, cm2_ref, cm4_ref, cm6_ref,
                  so_ref, do_ref):
    w = wlo_ref[...]

    def branch(pt):
        dt = t_ref[...] - pt
        d2 = dt * dt
        m2 = jnp.sum(d2, axis=1, keepdims=True)
        d4 = d2 * d2
        m4 = jnp.sum(d4, axis=1, keepdims=True)
        m6 = jnp.sum(d4 * d2, axis=1, keepdims=True)
        hi = (jnp.float32(L) + m2 * cm2_ref[...] + m4 * cm4_ref[...]
              + m6 * cm6_ref[...])
        lo = _poly_cos(dt[:, 0:1] * w)
        for l in range(1, L):
            lo = lo + _poly_cos(dt[:, l:l + 1] * w)
        return lo, hi

    lo, hi = branch(spt_ref[...])
    so_ref[:, 0:T_LO] = lo
    so_ref[:, T_LO:T_DIM] = hi
    lo, hi = branch(dpt_ref[...])
    do_ref[:, 0:T_LO] = lo
    do_ref[:, T_LO:T_DIM] = hi


@jax.jit
def _tc_time(t, spt, dpt, wlo, cm2, cm4, cm6):
    grid = (B // TBT,)
    return pl.pallas_call(
        _tc_time_body,
        grid=grid,
        in_specs=[
            pl.BlockSpec((TBT, 1), lambda i: (i, 0)),
            pl.BlockSpec((TBT, L), lambda i: (i, 0)),
            pl.BlockSpec((TBT, L), lambda i: (i, 0)),
            pl.BlockSpec((1, T_LO), lambda i: (0, 0)),
            pl.BlockSpec((1, T_DIM - T_LO), lambda i: (0, 0)),
            pl.BlockSpec((1, T_DIM - T_LO), lambda i: (0, 0)),
            pl.BlockSpec((1, T_DIM - T_LO), lambda i: (0, 0)),
        ],
        out_specs=[
            pl.BlockSpec((TBT, T_DIM), lambda i: (i, 0)),
            pl.BlockSpec((TBT, T_DIM), lambda i: (i, 0)),
        ],
        out_shape=[
            jax.ShapeDtypeStruct((B, T_DIM), jnp.float32),
            jax.ShapeDtypeStruct((B, T_DIM), jnp.float32),
        ],
    )(t, spt, dpt, wlo, cm2, cm4, cm6)


TBM = 1024


def _tc_mm_body(scr_ref, sns_ref, ses_ref, sts_ref,
                dcr_ref, dns_ref, des_ref, dts_ref,
                wc_ref, wn_ref, we_ref, wt_ref, b_ref,
                so_ref, do_ref):
    def branch(cr, ns, es, ts):
        acc = jnp.dot(cr, wc_ref[...], preferred_element_type=jnp.float32)
        acc = acc + jnp.dot(ns, wn_ref[...], preferred_element_type=jnp.float32)
        acc = acc + jnp.dot(es, we_ref[...], preferred_element_type=jnp.float32)
        acc = acc + jnp.dot(ts, wt_ref[...], preferred_element_type=jnp.float32)
        return jnp.maximum(acc + b_ref[...], 0.0)

    r2 = lambda x: x.reshape(TBM, D_FEAT)
    so_ref[...] = branch(r2(scr_ref[...]), r2(sns_ref[...]),
                         r2(ses_ref[...]), sts_ref[...])
    do_ref[...] = branch(r2(dcr_ref[...]), r2(dns_ref[...]),
                         r2(des_ref[...]), dts_ref[...])


@jax.jit
def _tc_mm(scr, sns, ses, sts, dcr, dns, des, dts, WcT, WnT, WeT, WtT, bias):
    grid = (B // TBM,)
    row = lambda i: (i, 0)
    fix = lambda i: (0, 0)
    return pl.pallas_call(
        _tc_mm_body,
        grid=grid,
        in_specs=[
            pl.BlockSpec((TBM // 8, 8, D_FEAT), lambda i: (i, 0, 0)),
            pl.BlockSpec((TBM // 8, 8, D_FEAT), lambda i: (i, 0, 0)),
            pl.BlockSpec((TBM // 8, 8, D_FEAT), lambda i: (i, 0, 0)),
            pl.BlockSpec((TBM, T_DIM), row),
            pl.BlockSpec((TBM // 8, 8, D_FEAT), lambda i: (i, 0, 0)),
            pl.BlockSpec((TBM // 8, 8, D_FEAT), lambda i: (i, 0, 0)),
            pl.BlockSpec((TBM // 8, 8, D_FEAT), lambda i: (i, 0, 0)),
            pl.BlockSpec((TBM, T_DIM), row),
            pl.BlockSpec((D_FEAT, D_FEAT), fix),
            pl.BlockSpec((D_FEAT, D_FEAT), fix),
            pl.BlockSpec((D_FEAT, D_FEAT), fix),
            pl.BlockSpec((T_DIM, D_FEAT), fix),
            pl.BlockSpec((1, D_FEAT), fix),
        ],
        out_specs=[
            pl.BlockSpec((TBM, D_FEAT), row),
            pl.BlockSpec((TBM, D_FEAT), row),
        ],
        out_shape=[
            jax.ShapeDtypeStruct((B, D_FEAT), jnp.float32),
            jax.ShapeDtypeStruct((B, D_FEAT), jnp.float32),
        ],
    )(scr, sns, ses, sts, dcr, dns, des, dts, WcT, WnT, WeT, WtT, bias)


def kernel(src_node_ids, dst_node_ids, node_interact_times,
           src_padded_nodes_neighbor_ids, src_padded_nodes_edge_ids, src_padded_nodes_neighbor_times,
           dst_padded_nodes_neighbor_ids, dst_padded_nodes_edge_ids, dst_padded_nodes_neighbor_times,
           node_raw_features, edge_raw_features, W, b):
    s_nbr = src_padded_nodes_neighbor_ids.reshape(-1).astype(jnp.int32)
    s_eid = src_padded_nodes_edge_ids.reshape(-1).astype(jnp.int32)
    d_nbr = dst_padded_nodes_neighbor_ids.reshape(-1).astype(jnp.int32)
    d_eid = dst_padded_nodes_edge_ids.reshape(-1).astype(jnp.int32)
    s_cur = src_node_ids.astype(jnp.int32)
    d_cur = dst_node_ids.astype(jnp.int32)
    t = node_interact_times[:, None]

    sns, ses, scr = _sc_gather(node_raw_features, edge_raw_features,
                               s_nbr, s_eid, s_cur)
    dns, des, dcr = _sc_gather(node_raw_features, edge_raw_features,
                               d_nbr, d_eid, d_cur)

    sts, dts = _tc_time(t, src_padded_nodes_neighbor_times,
                        dst_padded_nodes_neighbor_times,
                        jnp.asarray(_WLO), jnp.asarray(_CM2),
                        jnp.asarray(_CM4), jnp.asarray(_CM6))

    inv_l = jnp.float32(1.0 / L)
    WcT = W[:, :D_FEAT].T
    WnT = (W[:, D_FEAT:2 * D_FEAT] * inv_l).T
    WeT = jnp.concatenate(
        [(W[:, 2 * D_FEAT:2 * D_FEAT + D_EDGE] * inv_l).T,
         jnp.zeros((D_FEAT - D_EDGE, D_FEAT), jnp.float32)], axis=0)
    WtT = (W[:, 2 * D_FEAT + D_EDGE:] * inv_l).T
    bias = b[None, :]

    out_s, out_d = _tc_mm(scr, sns, ses, sts, dcr, dns, des, dts,
                          WcT, WnT, WeT, WtT, bias)
    return (out_s, out_d, jnp.zeros_like(out_s))
